# baseline (speedup 1.0000x reference)
import functools

import jax
import jax.numpy as jnp
from jax import lax
from jax.experimental import pallas as pl
from jax.experimental.pallas import tpu as pltpu
from jax.experimental.pallas import tpu_sc as plsc

NUM_EMBEDDINGS = 1000000
EMBEDDING_DIM = 64
TP_WORLD_SIZE = 2
NUM_EMB_PER_PART = NUM_EMBEDDINGS // TP_WORLD_SIZE
VOCAB_START = 0
VOCAB_END = NUM_EMB_PER_PART
NUM_TOKENS = 16384

NC = 2
NS = 16
NW = NC * NS
BPW = NUM_TOKENS // NW

_mesh = plsc.VectorSubcoreMesh(core_axis_name="c", subcore_axis_name="s")


@functools.partial(
    pl.kernel,
    mesh=_mesh,
    out_type=jax.ShapeDtypeStruct((NUM_TOKENS, EMBEDDING_DIM), jnp.float32),
    scratch_types=[
        pltpu.VMEM((BPW,), jnp.int32),
        pltpu.VMEM((BPW, EMBEDDING_DIM), jnp.float32),
        pltpu.SemaphoreType.DMA,
    ],
)
def _emb_kernel(x_hbm, w_hbm, z_hbm, out_hbm, idx_s, rows_v, sem):
    wid = lax.axis_index("s") * NC + lax.axis_index("c")
    base = wid * BPW

    pltpu.sync_copy(x_hbm.at[pl.ds(base, BPW)], idx_s)

    def issue(g, _):
        iv = idx_s[pl.ds(g * 16, 16)]
        for l in range(16):
            rel = iv[l] - VOCAB_START
            valid = (rel >= 0) & (rel < NUM_EMB_PER_PART)
            t = g * 16 + l

            @pl.when(valid)
            def _():
                pltpu.async_copy(
                    w_hbm.at[pl.ds(rel, 1)], rows_v.at[pl.ds(t, 1)], sem
                )

            @pl.when(jnp.logical_not(valid))
            def _():
                pltpu.async_copy(
                    z_hbm.at[pl.ds(0, 1)], rows_v.at[pl.ds(t, 1)], sem
                )

        return 0

    lax.fori_loop(0, BPW // 16, issue, 0)

    pltpu.make_async_copy(w_hbm.at[pl.ds(0, BPW)], rows_v, sem).wait()

    pltpu.sync_copy(rows_v, out_hbm.at[pl.ds(base, BPW)])


def kernel(x, weight):
    zeros = jnp.zeros((1, EMBEDDING_DIM), jnp.float32)
    return _emb_kernel(x.astype(jnp.int32), weight, zeros)
